# Initial kernel scaffold; baseline (speedup 1.0000x reference)
import jax
import jax.numpy as jnp
from jax.experimental import pallas as pl


def kernel(node_feats, edge_attrs, senders, receivers, W_up, W_msg, W_edge, W_down):
    raise NotImplementedError("write your pallas kernel here")


# speedup vs baseline: 2.6225x; 2.6225x over previous
import functools

import jax
import jax.numpy as jnp
from jax import lax
from jax.experimental import pallas as pl
from jax.experimental.pallas import tpu as pltpu
from jax.experimental.pallas import tpu_sc as plsc

N = 10000
E = 320000
D = 128
DE = 16
AVG_INV = 1.0 / 32.0

NC = 2
NS = 16
NW = NC * NS
EPT = E // NW
CH = 80
NCHUNK = EPT // CH
ROWS_PT = 624
ROWS_TAIL = N - NS * ROWS_PT
LANES = 16
CGRP = D // LANES

_EB = 4000


def _mm2_body(nf_ref, wu_ref, wm_ref, hm_ref):
    h = jnp.dot(nf_ref[...], wu_ref[...], preferred_element_type=jnp.float32)
    hm_ref[...] = jnp.dot(h, wm_ref[...], preferred_element_type=jnp.float32)


def _edgew_body(ea_ref, we_ref, ew_ref):
    ew_ref[...] = jnp.dot(ea_ref[...], we_ref[...],
                          preferred_element_type=jnp.float32)


def _down_body(parts_ref, wd_ref, out_ref):
    a = (parts_ref[:N, :] + parts_ref[N:, :]) * AVG_INV
    out_ref[...] = jnp.dot(a, wd_ref[...], preferred_element_type=jnp.float32)


def _sc_body(hm, ew, snd, rcv, zeros, out, sidx, ridx, rows_v, ew_v, agg_sh,
             sem):
    c = lax.axis_index("c")
    s = lax.axis_index("s")
    wid = s * NC + c
    row0 = s * ROWS_PT
    pltpu.sync_copy(zeros.at[pl.ds(row0, ROWS_PT)],
                    agg_sh.at[pl.ds(row0, ROWS_PT)])

    @pl.when(s == NS - 1)
    def _zero_tail():
        pltpu.sync_copy(zeros.at[pl.ds(NS * ROWS_PT, ROWS_TAIL)],
                        agg_sh.at[pl.ds(NS * ROWS_PT, ROWS_TAIL)])

    plsc.subcore_barrier()

    def chunk(j, carry):
        base = wid * EPT + j * CH
        pltpu.sync_copy(snd.at[pl.ds(base, CH)], sidx)
        pltpu.sync_copy(rcv.at[pl.ds(base, CH)], ridx)
        pltpu.async_copy(hm.at[sidx], rows_v, sem).wait()
        pltpu.sync_copy(ew.at[pl.ds(base, CH)], ew_v)

        def row(r, carry2):
            for k in range(CGRP):
                sl = pl.ds(k * LANES, LANES)
                v = rows_v[r, sl] * ew_v[r, sl]
                rows_v[r, sl] = jnp.maximum(v, 0.0)
            return carry2

        lax.fori_loop(0, CH, row, 0)
        pltpu.sync_copy(rows_v, agg_sh.at[ridx], add=True)
        return carry

    lax.fori_loop(0, NCHUNK, chunk, 0)
    plsc.subcore_barrier()
    pltpu.sync_copy(agg_sh.at[pl.ds(row0, ROWS_PT)],
                    out.at[pl.ds(c * N + row0, ROWS_PT)])

    @pl.when(s == NS - 1)
    def _write_tail():
        pltpu.sync_copy(agg_sh.at[pl.ds(NS * ROWS_PT, ROWS_TAIL)],
                        out.at[pl.ds(c * N + NS * ROWS_PT, ROWS_TAIL)])


_sc_scatter = functools.partial(
    pl.kernel,
    out_type=jax.ShapeDtypeStruct((NC * N, D), jnp.float32),
    mesh=plsc.VectorSubcoreMesh(core_axis_name="c", subcore_axis_name="s"),
    scratch_types=[
        pltpu.VMEM((CH,), jnp.int32),
        pltpu.VMEM((CH,), jnp.int32),
        pltpu.VMEM((CH, D), jnp.float32),
        pltpu.VMEM((CH, D), jnp.float32),
        pltpu.VMEM_SHARED((N, D), jnp.float32),
        pltpu.SemaphoreType.DMA,
    ],
)(_sc_body)


def kernel(node_feats, edge_attrs, senders, receivers, W_up, W_msg, W_edge,
           W_down):
    hm = pl.pallas_call(
        _mm2_body,
        out_shape=jax.ShapeDtypeStruct((N, D), jnp.float32),
    )(node_feats, W_up, W_msg)
    edge_w = pl.pallas_call(
        _edgew_body,
        grid=(E // _EB,),
        in_specs=[
            pl.BlockSpec((_EB, DE), lambda i: (i, 0)),
            pl.BlockSpec((DE, D), lambda i: (0, 0)),
        ],
        out_specs=pl.BlockSpec((_EB, D), lambda i: (i, 0)),
        out_shape=jax.ShapeDtypeStruct((E, D), jnp.float32),
    )(edge_attrs, W_edge)
    zeros = jnp.zeros((N, D), jnp.float32)
    parts = _sc_scatter(hm, edge_w, senders, receivers, zeros)
    out = pl.pallas_call(
        _down_body,
        out_shape=jax.ShapeDtypeStruct((N, D), jnp.float32),
    )(parts, W_down)
    return out


# speedup vs baseline: 4.5439x; 1.7327x over previous
import functools

import jax
import jax.numpy as jnp
from jax import lax
from jax.experimental import pallas as pl
from jax.experimental.pallas import tpu as pltpu
from jax.experimental.pallas import tpu_sc as plsc

N = 10000
E = 320000
D = 128
DE = 16
AVG_INV = 1.0 / 32.0

NC = 2
NS = 16
NW = NC * NS
EPT = E // NW
CH = 40
NCHUNK = EPT // CH
RB = 3
RI = 4
UNROLL = 12
NBLK = -(-NCHUNK // UNROLL)
ROWS_PT = 624
ROWS_TAIL = N - NS * ROWS_PT
LANES = 16
CGRP = D // LANES

_EB = 4000


def _mm2_body(nf_ref, wu_ref, wm_ref, hm_ref):
    h = jnp.dot(nf_ref[...], wu_ref[...], preferred_element_type=jnp.float32)
    hm_ref[...] = jnp.dot(h, wm_ref[...], preferred_element_type=jnp.float32)


def _edgew_body(ea_ref, we_ref, ew_ref):
    ew_ref[...] = jnp.dot(ea_ref[...], we_ref[...],
                          preferred_element_type=jnp.float32)


def _down_body(parts_ref, wd_ref, out_ref):
    a = (parts_ref[:N, :] + parts_ref[N:, :]) * AVG_INV
    out_ref[...] = jnp.dot(a, wd_ref[...], preferred_element_type=jnp.float32)


def _sc_body(hm, ew, snd, rcv, zeros, out, sidx, ridx, rows_v, ew_v, agg_sh,
             *sems):
    sem_in = sems[0:RB]
    sem_sc = sems[RB:2 * RB]
    sem_idx = sems[2 * RB:2 * RB + RI]
    c = lax.axis_index("c")
    s = lax.axis_index("s")
    wid = s * NC + c
    ebase = wid * EPT
    row0 = s * ROWS_PT
    pltpu.sync_copy(zeros.at[pl.ds(row0, ROWS_PT)],
                    agg_sh.at[pl.ds(row0, ROWS_PT)])

    @pl.when(s == NS - 1)
    def _zero_tail():
        pltpu.sync_copy(zeros.at[pl.ds(NS * ROWS_PT, ROWS_TAIL)],
                        agg_sh.at[pl.ds(NS * ROWS_PT, ROWS_TAIL)])

    plsc.subcore_barrier()

    def issue_idx(j, q):
        base = ebase + j * CH
        pltpu.async_copy(snd.at[pl.ds(base, CH)], sidx.at[q], sem_idx[q])
        pltpu.async_copy(rcv.at[pl.ds(base, CH)], ridx.at[q], sem_idx[q])

    def wait_idx(q):
        pltpu.make_async_copy(snd.at[pl.ds(0, CH)], sidx.at[q],
                              sem_idx[q]).wait()
        pltpu.make_async_copy(rcv.at[pl.ds(0, CH)], ridx.at[q],
                              sem_idx[q]).wait()

    def issue_in(j, q, p):
        base = ebase + j * CH
        pltpu.async_copy(hm.at[sidx.at[q]], rows_v.at[p], sem_in[p])
        pltpu.async_copy(ew.at[pl.ds(base, CH)], ew_v.at[p], sem_in[p])

    def wait_in(q, p):
        pltpu.make_async_copy(hm.at[sidx.at[q]], rows_v.at[p],
                              sem_in[p]).wait()
        pltpu.make_async_copy(ew.at[pl.ds(0, CH)], ew_v.at[p],
                              sem_in[p]).wait()

    def issue_scatter(q, p):
        pltpu.async_copy(rows_v.at[p], agg_sh.at[ridx.at[q]], sem_sc[p],
                         add=True)

    def wait_scatter(q, p):
        pltpu.make_async_copy(rows_v.at[p], agg_sh.at[ridx.at[q]],
                              sem_sc[p]).wait()

    def compute(p):
        def row(r, carry):
            for k in range(CGRP):
                sl = pl.ds(k * LANES, LANES)
                v = rows_v[p, r, sl] * ew_v[p, r, sl]
                rows_v[p, r, sl] = jnp.maximum(v, 0.0)
            return carry

        lax.fori_loop(0, CH, row, 0)

    issue_idx(0, 0)
    issue_idx(1, 1)
    issue_idx(2, 2)
    wait_idx(0)
    issue_in(0, 0, 0)
    wait_idx(1)
    issue_in(1, 1, 1)

    def block(jj, carry):
        for u in range(UNROLL):
            j = jj * UNROLL + u
            p = u % RB
            q = u % RI

            @pl.when(j < NCHUNK)
            def _main():
                wait_in(q, p)
                compute(p)
                issue_scatter(q, p)

            @pl.when((j >= 1) & (j <= NCHUNK))
            def _drain_prev():
                wait_scatter((u - 1) % RI, (u - 1) % RB)

            @pl.when(j + 3 < NCHUNK)
            def _prefetch_idx():
                issue_idx(j + 3, (u + 3) % RI)

            @pl.when(j + 2 < NCHUNK)
            def _prefetch_data():
                wait_idx((u + 2) % RI)
                issue_in(j + 2, (u + 2) % RI, (u + 2) % RB)

        return carry

    lax.fori_loop(0, NBLK, block, 0)
    plsc.subcore_barrier()
    pltpu.sync_copy(agg_sh.at[pl.ds(row0, ROWS_PT)],
                    out.at[pl.ds(c * N + row0, ROWS_PT)])

    @pl.when(s == NS - 1)
    def _write_tail():
        pltpu.sync_copy(agg_sh.at[pl.ds(NS * ROWS_PT, ROWS_TAIL)],
                        out.at[pl.ds(c * N + NS * ROWS_PT, ROWS_TAIL)])


_sc_scatter = functools.partial(
    pl.kernel,
    out_type=jax.ShapeDtypeStruct((NC * N, D), jnp.float32),
    mesh=plsc.VectorSubcoreMesh(core_axis_name="c", subcore_axis_name="s"),
    scratch_types=[
        pltpu.VMEM((RI, CH), jnp.int32),
        pltpu.VMEM((RI, CH), jnp.int32),
        pltpu.VMEM((RB, CH, D), jnp.float32),
        pltpu.VMEM((RB, CH, D), jnp.float32),
        pltpu.VMEM_SHARED((N, D), jnp.float32),
    ] + [pltpu.SemaphoreType.DMA] * (2 * RB + RI),
)(_sc_body)


def kernel(node_feats, edge_attrs, senders, receivers, W_up, W_msg, W_edge,
           W_down):
    hm = pl.pallas_call(
        _mm2_body,
        out_shape=jax.ShapeDtypeStruct((N, D), jnp.float32),
    )(node_feats, W_up, W_msg)
    edge_w = pl.pallas_call(
        _edgew_body,
        grid=(E // _EB,),
        in_specs=[
            pl.BlockSpec((_EB, DE), lambda i: (i, 0)),
            pl.BlockSpec((DE, D), lambda i: (0, 0)),
        ],
        out_specs=pl.BlockSpec((_EB, D), lambda i: (i, 0)),
        out_shape=jax.ShapeDtypeStruct((E, D), jnp.float32),
    )(edge_attrs, W_edge)
    zeros = jnp.zeros((N, D), jnp.float32)
    parts = _sc_scatter(hm, edge_w, senders, receivers, zeros)
    out = pl.pallas_call(
        _down_body,
        out_shape=jax.ShapeDtypeStruct((N, D), jnp.float32),
    )(parts, W_down)
    return out
